# baseline (speedup 1.0000x reference)
import functools

import jax
import jax.numpy as jnp
from jax.experimental import pallas as pl
from jax.experimental.pallas import tpu as pltpu

_G = 128
_D = 32
_C = 256


def _compute(x2d, l_ref, l2_ref, bwa_ref, bw1_ref, bw2_ref, bias_ref):
    g = x2d.shape[0]
    n = l_ref.shape[0]
    xb = x2d.astype(jnp.bfloat16).reshape(g, n, _D)
    x0b = jnp.concatenate([xb[i] for i in range(g)], axis=1)
    x1b = jnp.dot(l_ref[...], x0b,
                  preferred_element_type=jnp.float32).astype(jnp.bfloat16)
    zb = jnp.dot(l2_ref[...], x1b,
                 preferred_element_type=jnp.float32).astype(jnp.bfloat16)
    bias = bias_ref[...]
    outs = []
    for j in range(g * _D // _C):
        sl = slice(j * _C, (j + 1) * _C)
        acc = jnp.dot(x0b[:, sl], bwa_ref[...],
                      preferred_element_type=jnp.float32)
        acc += jnp.dot(x1b[:, sl], bw1_ref[...],
                       preferred_element_type=jnp.float32)
        acc += jnp.dot(zb[:, sl], bw2_ref[...],
                       preferred_element_type=jnp.float32)
        outs.append((acc + bias).astype(jnp.bfloat16))
    out_t = jnp.concatenate(outs, axis=1)
    return jnp.stack(
        [out_t[:, i * _D:(i + 1) * _D] for i in range(g)],
        axis=0).reshape(g, n * _D).astype(jnp.float32)


def _body(y_hbm, l_ref, l2_ref, bwa_ref, bw1_ref, bw2_ref, bias_ref,
          out_hbm, xin, xout, in_sems, out_sems):
    i = pl.program_id(0)
    nsteps = pl.num_programs(0)
    g = _G

    def in_copy(step, sl_):
        return pltpu.make_async_copy(
            y_hbm.at[pl.ds(step * g, g), :], xin.at[sl_], in_sems.at[sl_])

    def out_copy(step, sl_):
        return pltpu.make_async_copy(
            xout.at[sl_], out_hbm.at[pl.ds(step * g, g), :],
            out_sems.at[sl_])

    slot = jax.lax.rem(i, 2)
    nslot = jax.lax.rem(i + 1, 2)

    @pl.when(i == 0)
    def _():
        in_copy(i, slot).start()

    @pl.when(i + 1 < nsteps)
    def _():
        in_copy(i + 1, nslot).start()

    in_copy(i, slot).wait()

    @pl.when(i >= 2)
    def _():
        out_copy(i - 2, slot).wait()

    xout[slot] = _compute(xin[slot], l_ref, l2_ref, bwa_ref, bw1_ref,
                          bw2_ref, bias_ref)
    out_copy(i, slot).start()

    @pl.when(i == nsteps - 1)
    def _():
        @pl.when(i >= 1)
        def _():
            out_copy(i - 1, nslot).wait()

        out_copy(i, slot).wait()


@functools.partial(jax.jit, static_argnums=0)
def _run(g, y, lmat, l2mat, bwa, bw1, bw2, bias2d):
    b, f = y.shape
    return pl.pallas_call(
        _body,
        grid=(b // g,),
        in_specs=[
            pl.BlockSpec(memory_space=pl.ANY),
            pl.BlockSpec(lmat.shape, lambda i: (0, 0)),
            pl.BlockSpec(l2mat.shape, lambda i: (0, 0)),
            pl.BlockSpec((_C, _C), lambda i: (0, 0)),
            pl.BlockSpec((_C, _C), lambda i: (0, 0)),
            pl.BlockSpec((_C, _C), lambda i: (0, 0)),
            pl.BlockSpec((1, _C), lambda i: (0, 0)),
        ],
        out_specs=pl.BlockSpec(memory_space=pl.ANY),
        out_shape=jax.ShapeDtypeStruct((b, f), jnp.float32),
        scratch_shapes=[
            pltpu.VMEM((2, g, f), jnp.float32),
            pltpu.VMEM((2, g, f), jnp.float32),
            pltpu.SemaphoreType.DMA((2,)),
            pltpu.SemaphoreType.DMA((2,)),
        ],
    )(y, lmat, l2mat, bwa, bw1, bw2, bias2d)


def kernel(t_local, y, L, W, b):
    del t_local
    d = W.shape[1]
    m = W.shape[0] // d
    eye = jnp.eye(_C // d, dtype=jnp.float32)
    w0, w1, w2 = (-0.1 * W[k::m, :] for k in range(m))
    bwa = jnp.kron(eye, w0 - w2).astype(jnp.bfloat16)
    bw1 = jnp.kron(eye, w1).astype(jnp.bfloat16)
    bw2 = jnp.kron(eye, w2).astype(jnp.bfloat16)
    bias2d = jnp.tile(-0.1 * b, _C // d).reshape(1, _C)
    return _run(_G, y, L.astype(jnp.bfloat16),
                (2.0 * L).astype(jnp.bfloat16), bwa, bw1, bw2, bias2d)


# speedup vs baseline: 1.0063x; 1.0063x over previous
import functools

import jax
import jax.numpy as jnp
from jax.experimental import pallas as pl

_G = 128
_D = 32
_C = 256


def _body(y_ref, l_ref, l2_ref, bwa_ref, bw1_ref, bw2_ref, bias_ref,
          out_ref):
    g = y_ref.shape[0]
    n = l_ref.shape[0]
    xb = y_ref[...].astype(jnp.bfloat16).reshape(g, n, _D)
    x0b = jnp.concatenate([xb[i] for i in range(g)], axis=1)
    x1b = jnp.dot(l_ref[...], x0b,
                  preferred_element_type=jnp.float32).astype(jnp.bfloat16)
    zb = jnp.dot(l2_ref[...], x1b,
                 preferred_element_type=jnp.float32).astype(jnp.bfloat16)
    bias = bias_ref[...]
    outs = []
    for j in range(g * _D // _C):
        sl = slice(j * _C, (j + 1) * _C)
        acc = jnp.dot(x0b[:, sl], bwa_ref[...],
                      preferred_element_type=jnp.float32)
        acc += jnp.dot(x1b[:, sl], bw1_ref[...],
                       preferred_element_type=jnp.float32)
        acc += jnp.dot(zb[:, sl], bw2_ref[...],
                       preferred_element_type=jnp.float32)
        outs.append((acc + bias).astype(jnp.bfloat16))
    out_t = jnp.concatenate(outs, axis=1)
    out_ref[...] = jnp.stack(
        [out_t[:, i * _D:(i + 1) * _D] for i in range(g)],
        axis=0).reshape(g, n * _D).astype(jnp.float32)


@functools.partial(jax.jit, static_argnums=0)
def _run(g, y, lmat, l2mat, bwa, bw1, bw2, bias2d):
    b, f = y.shape
    return pl.pallas_call(
        _body,
        grid=(b // g,),
        in_specs=[
            pl.BlockSpec((g, f), lambda i: (i, 0)),
            pl.BlockSpec(lmat.shape, lambda i: (0, 0)),
            pl.BlockSpec(l2mat.shape, lambda i: (0, 0)),
            pl.BlockSpec((_C, _C), lambda i: (0, 0)),
            pl.BlockSpec((_C, _C), lambda i: (0, 0)),
            pl.BlockSpec((_C, _C), lambda i: (0, 0)),
            pl.BlockSpec((1, _C), lambda i: (0, 0)),
        ],
        out_specs=pl.BlockSpec((g, f), lambda i: (i, 0)),
        out_shape=jax.ShapeDtypeStruct((b, f), jnp.float32),
    )(y, lmat, l2mat, bwa, bw1, bw2, bias2d)


def kernel(t_local, y, L, W, b):
    del t_local
    d = W.shape[1]
    m = W.shape[0] // d
    eye = jnp.eye(_C // d, dtype=jnp.float32)
    w0, w1, w2 = (-0.1 * W[k::m, :] for k in range(m))
    bwa = jnp.kron(eye, w0 - w2).astype(jnp.bfloat16)
    bw1 = jnp.kron(eye, w1).astype(jnp.bfloat16)
    bw2 = jnp.kron(eye, w2).astype(jnp.bfloat16)
    bias2d = jnp.tile(-0.1 * b, _C // d).reshape(1, _C)
    return _run(_G, y, L.astype(jnp.bfloat16),
                (2.0 * L).astype(jnp.bfloat16), bwa, bw1, bw2, bias2d)
